# baseline (speedup 1.0000x reference)
import functools

import jax
import jax.numpy as jnp
from jax import lax
from jax.experimental import pallas as pl
from jax.experimental.pallas import tpu as pltpu
from jax.experimental.pallas import tpu_sc as plsc

GRID = 64
D_ROW = 1024
D_COL = 1024
D_MODEL = D_ROW + D_COL
SEQ = GRID * GRID

NC = 2
NS = 16
NW = NC * NS
ROWS_PER_W = SEQ // NW
CH = 16
NCH = ROWS_PER_W // CH
NSLOT = 3


@functools.partial(
    pl.kernel,
    mesh=plsc.VectorSubcoreMesh(core_axis_name="c", subcore_axis_name="s"),
    out_type=jax.ShapeDtypeStruct((SEQ, D_MODEL), jnp.float32),
    scratch_types=(
        [pltpu.VMEM((CH, D_MODEL), jnp.float32)] * NSLOT
        + [pltpu.SemaphoreType.DMA] * (2 * NSLOT)
    ),
)
def _pos_enc_sc(row_hbm, col_hbm, out_hbm, buf0, buf1, buf2,
                sl0, sl1, sl2, sw0, sw1, sw2):
    bufs = (buf0, buf1, buf2)
    sem_ld = (sl0, sl1, sl2)
    sem_wr = (sw0, sw1, sw2)
    wid = lax.axis_index("s") * NC + lax.axis_index("c")
    w0 = pl.multiple_of(wid * ROWS_PER_W, ROWS_PER_W)

    def issue_loads(k, s):
        base = pl.multiple_of(w0 + k * CH, CH)
        r = base >> 6
        c0 = base & (GRID - 1)
        hs = []
        for i in range(CH):
            hs.append(pltpu.async_copy(
                row_hbm.at[pl.ds(r, 1)],
                bufs[s].at[pl.ds(i, 1), pl.ds(0, D_ROW)], sem_ld[s]))
            hs.append(pltpu.async_copy(
                col_hbm.at[pl.ds(c0 + i, 1)],
                bufs[s].at[pl.ds(i, 1), pl.ds(D_ROW, D_COL)], sem_ld[s]))
        return hs

    def issue_write(k, s):
        base = pl.multiple_of(w0 + k * CH, CH)
        return pltpu.async_copy(bufs[s], out_hbm.at[pl.ds(base, CH)], sem_wr[s])

    wh = {}
    ldh = {}
    for k in range(NCH):
        s = k % NSLOT
        if k >= NSLOT:
            wh[k - NSLOT].wait()
        ldh[k] = issue_loads(k, s)
        if k >= 1:
            for h in ldh[k - 1]:
                h.wait()
            wh[k - 1] = issue_write(k - 1, (k - 1) % NSLOT)
    for h in ldh[NCH - 1]:
        h.wait()
    wh[NCH - 1] = issue_write(NCH - 1, (NCH - 1) % NSLOT)
    for k in range(NCH - NSLOT, NCH):
        wh[k].wait()


def kernel(seq_len, row_embed, col_embed):
    del seq_len
    return _pos_enc_sc(row_embed, col_embed)


# speedup vs baseline: 1.3246x; 1.3246x over previous
import functools

import jax
import jax.numpy as jnp
from jax import lax
from jax.experimental import pallas as pl
from jax.experimental.pallas import tpu as pltpu
from jax.experimental.pallas import tpu_sc as plsc

GRID = 64
D_ROW = 1024
D_COL = 1024
D_MODEL = D_ROW + D_COL
SEQ = GRID * GRID

NC = 2
NS = 16
NW = NC * NS
ROWS_PER_W = SEQ // NW
CH = 16
NCH = ROWS_PER_W // CH
NSLOT = 3


@functools.partial(
    pl.kernel,
    mesh=plsc.VectorSubcoreMesh(core_axis_name="c", subcore_axis_name="s"),
    out_type=jax.ShapeDtypeStruct((SEQ, D_MODEL), jnp.float32),
    scratch_types=(
        [pltpu.VMEM((2, D_ROW), jnp.float32)]
        + [pltpu.VMEM((CH, D_MODEL), jnp.float32)] * NSLOT
        + [pltpu.SemaphoreType.DMA] * (2 * NSLOT)
    ),
)
def _pos_enc_sc(row_hbm, col_hbm, out_hbm, rowbuf, buf0, buf1, buf2,
                sl0, sl1, sl2, sw0, sw1, sw2):
    bufs = (buf0, buf1, buf2)
    sem_ld = (sl0, sl1, sl2)
    sem_wr = (sw0, sw1, sw2)
    wid = lax.axis_index("s") * NC + lax.axis_index("c")
    w0 = pl.multiple_of(wid * ROWS_PER_W, ROWS_PER_W)

    pltpu.sync_copy(row_hbm.at[pl.ds(2 * wid, 1)], rowbuf.at[pl.ds(0, 1)])
    pltpu.sync_copy(row_hbm.at[pl.ds(2 * wid + 1, 1)], rowbuf.at[pl.ds(1, 1)])

    def make_fill(t, s):
        def fill(j, _):
            off = pl.multiple_of(j * 16, 16)
            v = rowbuf[t, pl.ds(off, 16)]
            for i in range(CH):
                bufs[s][i, pl.ds(off, 16)] = v
            return 0
        return fill

    wh = {}
    for k in range(NCH):
        s = k % NSLOT
        t = k // (NCH // 2)
        c0 = CH * (k % (NCH // 2))
        base = pl.multiple_of(w0 + k * CH, CH)
        if k >= NSLOT:
            wh[k - NSLOT].wait()
        cl = pltpu.async_copy(
            col_hbm.at[pl.ds(c0, CH)],
            bufs[s].at[:, pl.ds(D_ROW, D_COL)], sem_ld[s])
        lax.fori_loop(0, D_ROW // 16, make_fill(t, s), 0)
        cl.wait()
        wh[k] = pltpu.async_copy(bufs[s], out_hbm.at[pl.ds(base, CH)],
                                 sem_wr[s])
    for k in range(NCH - NSLOT, NCH):
        wh[k].wait()


def kernel(seq_len, row_embed, col_embed):
    del seq_len
    return _pos_enc_sc(row_embed, col_embed)


# speedup vs baseline: 1.6498x; 1.2454x over previous
import functools

import jax
import jax.numpy as jnp
from jax import lax
from jax.experimental import pallas as pl
from jax.experimental.pallas import tpu as pltpu
from jax.experimental.pallas import tpu_sc as plsc

GRID = 64
D_ROW = 1024
D_COL = 1024
D_MODEL = D_ROW + D_COL
SEQ = GRID * GRID

NC = 2
NS = 16
NW = NC * NS
HB = GRID // 2
QB = GRID // 4


@functools.partial(
    pl.kernel,
    mesh=plsc.VectorSubcoreMesh(core_axis_name="c", subcore_axis_name="s"),
    out_type=jax.ShapeDtypeStruct((SEQ, D_MODEL), jnp.float32),
    scratch_types=[
        pltpu.VMEM((2, D_ROW), jnp.float32),
        pltpu.VMEM((QB, D_ROW), jnp.float32),
        pltpu.VMEM((QB, D_ROW), jnp.float32),
        pltpu.VMEM((HB, D_COL), jnp.float32),
        pltpu.VMEM((HB, D_COL), jnp.float32),
        pltpu.SemaphoreType.DMA,
        pltpu.SemaphoreType.DMA,
        pltpu.SemaphoreType.DMA,
    ],
)
def _pos_enc_sc(row_hbm, col_hbm, out_hbm, rowbuf, left_a, left_b,
                col_a, col_b, sem_c, sem_lw, sem_rw):
    lefts = (left_a, left_b)
    wid = lax.axis_index("s") * NC + lax.axis_index("c")

    cp_a = pltpu.async_copy(col_hbm.at[pl.ds(0, HB)], col_a, sem_c)
    cp_b = pltpu.async_copy(col_hbm.at[pl.ds(HB, HB)], col_b, sem_c)
    pltpu.sync_copy(row_hbm.at[pl.ds(2 * wid, 1)], rowbuf.at[pl.ds(0, 1)])
    pltpu.sync_copy(row_hbm.at[pl.ds(2 * wid + 1, 1)], rowbuf.at[pl.ds(1, 1)])

    def make_fill(t):
        def fill(j, _):
            off = pl.multiple_of(j * 16, 16)
            v = rowbuf[t, pl.ds(off, 16)]
            for i in range(QB):
                lefts[t][i, pl.ds(off, 16)] = v
            return 0
        return fill

    writes = []
    for t in range(2):
        rbase = pl.multiple_of((2 * wid + t) * GRID, GRID)
        lax.fori_loop(0, D_ROW // 16, make_fill(t), 0)
        for q in range(4):
            writes.append(pltpu.async_copy(
                lefts[t],
                out_hbm.at[pl.ds(rbase + q * QB, QB), pl.ds(0, D_ROW)],
                sem_lw))
    cp_a.wait()
    cp_b.wait()
    for t in range(2):
        rbase = pl.multiple_of((2 * wid + t) * GRID, GRID)
        writes.append(pltpu.async_copy(
            col_a, out_hbm.at[pl.ds(rbase, HB), pl.ds(D_ROW, D_COL)],
            sem_rw))
        writes.append(pltpu.async_copy(
            col_b, out_hbm.at[pl.ds(rbase + HB, HB), pl.ds(D_ROW, D_COL)],
            sem_rw))
    for w in writes:
        w.wait()


def kernel(seq_len, row_embed, col_embed):
    del seq_len
    return _pos_enc_sc(row_embed, col_embed)


# speedup vs baseline: 2.1613x; 1.3101x over previous
import functools

import jax
import jax.numpy as jnp
from jax import lax
from jax.experimental import pallas as pl
from jax.experimental.pallas import tpu as pltpu
from jax.experimental.pallas import tpu_sc as plsc

GRID = 64
D_ROW = 1024
D_COL = 1024
D_MODEL = D_ROW + D_COL
SEQ = GRID * GRID

NC = 2
NS = 16
NW = NC * NS
HB = GRID // 2


@functools.partial(
    pl.kernel,
    mesh=plsc.VectorSubcoreMesh(core_axis_name="c", subcore_axis_name="s"),
    out_type=jax.ShapeDtypeStruct((SEQ, D_MODEL), jnp.float32),
    scratch_types=[
        pltpu.VMEM((1, D_ROW), jnp.float32),
        pltpu.VMEM((HB, D_ROW), jnp.float32),
        pltpu.VMEM((HB, D_COL), jnp.float32),
        pltpu.VMEM((HB, D_COL), jnp.float32),
        pltpu.VMEM_SHARED((GRID, D_COL), jnp.float32),
        pltpu.SemaphoreType.DMA,
        pltpu.SemaphoreType.DMA,
        pltpu.SemaphoreType.DMA,
        pltpu.SemaphoreType.DMA,
    ],
)
def _pos_enc_sc(row_hbm, col_hbm, out_hbm, rowbuf, left, col_a, col_b,
                col_sh, sem_s, sem_c, sem_lw, sem_rw):
    sid = lax.axis_index("s")
    wid = sid * NC + lax.axis_index("c")

    @pl.when(sid == 0)
    def _():
        pltpu.sync_copy(col_hbm, col_sh)
    plsc.subcore_barrier()
    cp_a = pltpu.async_copy(col_sh.at[pl.ds(0, HB)], col_a, sem_c)
    cp_b = pltpu.async_copy(col_sh.at[pl.ds(HB, HB)], col_b, sem_c)

    def replicate(j, _):
        off = pl.multiple_of(j * 16, 16)
        v = rowbuf[0, pl.ds(off, 16)]
        for i in range(HB):
            left[i, pl.ds(off, 16)] = v
        return 0

    right_writes = []
    for t in range(2):
        r = 2 * wid + t
        rbase = pl.multiple_of(r * GRID, GRID)
        pltpu.sync_copy(row_hbm.at[pl.ds(r, 1)], rowbuf)
        lax.fori_loop(0, D_ROW // 16, replicate, 0)
        wl0 = pltpu.async_copy(
            left, out_hbm.at[pl.ds(rbase, HB), pl.ds(0, D_ROW)], sem_lw)
        wl1 = pltpu.async_copy(
            left, out_hbm.at[pl.ds(rbase + HB, HB), pl.ds(0, D_ROW)], sem_lw)
        if t == 0:
            cp_a.wait()
            cp_b.wait()
        right_writes.append(pltpu.async_copy(
            col_a, out_hbm.at[pl.ds(rbase, HB), pl.ds(D_ROW, D_COL)], sem_rw))
        right_writes.append(pltpu.async_copy(
            col_b, out_hbm.at[pl.ds(rbase + HB, HB), pl.ds(D_ROW, D_COL)],
            sem_rw))
        wl0.wait()
        wl1.wait()
    for w in right_writes:
        w.wait()


def kernel(seq_len, row_embed, col_embed):
    del seq_len
    return _pos_enc_sc(row_embed, col_embed)
